# Initial kernel scaffold; baseline (speedup 1.0000x reference)
import jax
import jax.numpy as jnp
from jax.experimental import pallas as pl


def kernel(x, params, edge_index):
    raise NotImplementedError("write your pallas kernel here")


# speedup vs baseline: 13.8033x; 13.8033x over previous
import functools

import numpy as np
import jax
import jax.numpy as jnp
from jax import lax
from jax.experimental import pallas as pl
from jax.experimental.pallas import tpu as pltpu
from jax.experimental.pallas import tpu_sc as plsc

N = 10000
OUT_C = 2
BN_EPS = 1e-5
GCN_DIMS = [128, 64, 32]
GAT_DIMS = [128, 64, 32]
GAT_HEADS = [8, 4, 1]
BLK = 1000
_BN_INV = float(1.0 / np.sqrt(1.0 + BN_EPS))

E_RAW = 160000
EV = E_RAW + N
EP = 172032


def _mm(x, w, b, relu=False, row_scale=None):
    n, k = x.shape
    f = w.shape[1]
    scaled = row_scale is not None

    def body(*refs):
        if scaled:
            x_ref, w_ref, b_ref, s_ref, o_ref = refs
        else:
            x_ref, w_ref, b_ref, o_ref = refs
        y = jnp.dot(x_ref[...], w_ref[...],
                    preferred_element_type=jnp.float32) + b_ref[...]
        if scaled:
            y = y * s_ref[...]
        o_ref[...] = jnp.maximum(y, 0.0) if relu else y

    in_specs = [
        pl.BlockSpec((BLK, k), lambda i: (i, 0)),
        pl.BlockSpec((k, f), lambda i: (0, 0)),
        pl.BlockSpec((1, f), lambda i: (0, 0)),
    ]
    args = [x, w, b.reshape(1, f)]
    if scaled:
        in_specs.append(pl.BlockSpec((BLK, 1), lambda i: (i, 0)))
        args.append(row_scale)
    return pl.pallas_call(
        body,
        grid=(n // BLK,),
        in_specs=in_specs,
        out_specs=pl.BlockSpec((BLK, f), lambda i: (i, 0)),
        out_shape=jax.ShapeDtypeStruct((n, f), jnp.float32),
    )(*args)


def _gcn_post(scat, dis, bconv, g, bbn, res):
    f = scat.shape[1]

    def body(s_ref, d_ref, bc_ref, g_ref, bb_ref, r_ref, o_ref):
        t = (s_ref[...] * d_ref[...] + bc_ref[...]) * (g_ref[...] * _BN_INV) \
            + bb_ref[...]
        o_ref[...] = jnp.maximum(t, 0.0) + r_ref[...]

    return pl.pallas_call(
        body,
        grid=(N // BLK,),
        in_specs=[
            pl.BlockSpec((BLK, f), lambda i: (i, 0)),
            pl.BlockSpec((BLK, 1), lambda i: (i, 0)),
            pl.BlockSpec((1, f), lambda i: (0, 0)),
            pl.BlockSpec((1, f), lambda i: (0, 0)),
            pl.BlockSpec((1, f), lambda i: (0, 0)),
            pl.BlockSpec((BLK, f), lambda i: (i, 0)),
        ],
        out_specs=pl.BlockSpec((BLK, f), lambda i: (i, 0)),
        out_shape=jax.ShapeDtypeStruct((N, f), jnp.float32),
    )(scat, dis, bconv.reshape(1, f), g.reshape(1, f), bbn.reshape(1, f), res)


def _gat_post(scat, rdn, bias, g, bbn, heads):
    hf = scat.shape[1]
    f = hf // heads

    def body(s_ref, rd_ref, b_ref, g_ref, bb_ref, o_ref):
        s = s_ref[...].reshape(BLK, heads, f) * rd_ref[...].reshape(BLK, heads, 1)
        t = (s.reshape(BLK, hf) + b_ref[...]) * (g_ref[...] * _BN_INV) + bb_ref[...]
        o_ref[...] = jnp.maximum(t, 0.0)

    return pl.pallas_call(
        body,
        grid=(N // BLK,),
        in_specs=[
            pl.BlockSpec((BLK, hf), lambda i: (i, 0)),
            pl.BlockSpec((BLK, heads), lambda i: (i, 0)),
            pl.BlockSpec((1, hf), lambda i: (0, 0)),
            pl.BlockSpec((1, hf), lambda i: (0, 0)),
            pl.BlockSpec((1, hf), lambda i: (0, 0)),
        ],
        out_specs=pl.BlockSpec((BLK, hf), lambda i: (i, 0)),
        out_shape=jax.ShapeDtypeStruct((N, hf), jnp.float32),
    )(scat, rdn, bias.reshape(1, hf), g.reshape(1, hf), bbn.reshape(1, hf))


def _shift(a_s, a_d):
    h = a_s.shape[1]

    def body(s_ref, d_ref, o_ref):
        m = (jnp.max(s_ref[...], axis=0, keepdims=True)
             + jnp.max(d_ref[...], axis=0, keepdims=True))
        o_ref[...] = jnp.maximum(m, 0.2 * m)

    return pl.pallas_call(
        body, out_shape=jax.ShapeDtypeStruct((1, h), jnp.float32))(a_s, a_d)


def _reduce_sc(p, mode, corr=None):
    m = p.shape[1]
    has_corr = corr is not None

    def body(*refs):
        if has_corr:
            p_ref, c_ref, o_ref = refs
        else:
            p_ref, o_ref = refs
        s = jnp.sum(p_ref[...], axis=0, keepdims=True)
        if has_corr:
            s = s - c_ref[...]
        if mode == "rsqrt":
            r = lax.rsqrt(s)
        else:
            r = 1.0 / (s + 1e-16)
        o_ref[...] = r

    args = (p, corr) if has_corr else (p,)
    return pl.pallas_call(
        body, out_shape=jax.ShapeDtypeStruct((1, m), jnp.float32))(*args)


def _final(hg, ha, ens_w):
    def body(g_ref, a_ref, w_ref, o_ref):
        def lsm(t):
            mx = jnp.max(t, axis=1, keepdims=True)
            return t - (mx + jnp.log(jnp.sum(jnp.exp(t - mx), axis=1,
                                             keepdims=True)))
        wv = w_ref[...]
        wv = jnp.exp(wv - jnp.max(wv))
        wv = wv / jnp.sum(wv)
        o_ref[...] = wv[:, 0:1] * lsm(g_ref[...]) + wv[:, 1:2] * lsm(a_ref[...])

    return pl.pallas_call(
        body, out_shape=jax.ShapeDtypeStruct((N, OUT_C), jnp.float32),
    )(hg, ha, ens_w)


_SC_MESH = plsc.VectorSubcoreMesh(core_axis_name="c", subcore_axis_name="s")
_SC_PARAMS = pltpu.CompilerParams(needs_layout_passes=False,
                                  use_tc_tiling_on_sc=False)
_EPT16 = EP // 16
_EPT32 = EP // 32
NPAD = 10240
_NROW = NPAD // 16


def _bc16(x):
    return lax.broadcast_in_dim(x, (16,), ())


def _zero_ref(ref, nrows, w):
    z = jnp.zeros((16,), jnp.float32)

    def bd(r, _):
        for j in range(w // 16):
            ref[r, pl.ds(j * 16, 16)] = z
        return 0

    lax.fori_loop(0, nrows, bd, 0)


def _zero_acc_rows(acc, zbuf, row0, ncopies):
    def bd(k, _):
        pltpu.sync_copy(zbuf, acc.at[pl.ds(row0 + k * 128, 128)])
        return 0

    lax.fori_loop(0, ncopies, bd, 0)


@functools.lru_cache(maxsize=None)
def _deg_kernel():
    @functools.partial(
        pl.kernel, mesh=_SC_MESH, compiler_params=_SC_PARAMS,
        out_type=jax.ShapeDtypeStruct((2, NPAD, 16), jnp.float32),
        scratch_types=[
            pltpu.VMEM((_EPT32,), jnp.int32),
            pltpu.VMEM((128, 16), jnp.float32),
            pltpu.VMEM((128, 16), jnp.float32),
            pltpu.VMEM((128,), jnp.int32),
            pltpu.VMEM_SHARED((NPAD, 16), jnp.float32),
        ])
    def k(dst_hbm, out_hbm, dst_v, buf, zbuf, sidx_v, acc):
        cid = lax.axis_index("c")
        sid = lax.axis_index("s")
        wid = sid * 2 + cid
        iota = lax.iota(jnp.int32, 16)
        pltpu.sync_copy(dst_hbm.at[pl.ds(wid * _EPT32, _EPT32)], dst_v)
        one0 = jnp.where(iota < 1, jnp.float32(1.0), jnp.float32(0.0))
        def fill(r, _):
            buf[r, :] = one0
            return 0
        lax.fori_loop(0, 128, fill, 0)
        _zero_ref(zbuf, 128, 16)
        _zero_acc_rows(acc, zbuf, sid * _NROW, 5)
        plsc.subcore_barrier()

        def batch(b, _):
            e0 = b * 128
            for g in range(8):
                d16 = dst_v[pl.ds(e0 + g * 16, 16)]
                sidx_v[pl.ds(g * 16, 16)] = d16
            pltpu.sync_copy(buf, acc.at[sidx_v], add=True)
            return 0

        lax.fori_loop(0, _EPT32 // 128, batch, 0)
        plsc.subcore_barrier()

        def drain(k2, _):
            r0 = sid * _NROW + k2 * 128
            pltpu.sync_copy(acc.at[pl.ds(r0, 128)],
                            out_hbm.at[cid, pl.ds(r0, 128)])
            return 0

        lax.fori_loop(0, 5, drain, 0)

    return k


@functools.lru_cache(maxsize=None)
def _gcn_scatter_kernel(fc):
    nb = _EPT16 // 128

    @functools.partial(
        pl.kernel, mesh=_SC_MESH, compiler_params=_SC_PARAMS,
        out_type=jax.ShapeDtypeStruct((2, NPAD, fc), jnp.float32),
        scratch_types=[
            pltpu.VMEM((_EPT16,), jnp.int32),
            pltpu.VMEM((_EPT16,), jnp.int32),
            pltpu.VMEM((128,), jnp.int32),
            pltpu.VMEM((128,), jnp.int32),
            pltpu.VMEM((128, fc), jnp.float32),
            pltpu.VMEM((128, fc), jnp.float32),
            pltpu.VMEM_SHARED((NPAD, fc), jnp.float32),
        ])
    def k(table_hbm, src_hbm, dst_hbm, out_hbm,
          src_v, dst_v, gidx_v, sidx_v, rows_v, zbuf, acc):
        cid = lax.axis_index("c")
        sid = lax.axis_index("s")
        base = sid * _EPT16
        pltpu.sync_copy(src_hbm.at[pl.ds(base, _EPT16)], src_v)
        pltpu.sync_copy(dst_hbm.at[pl.ds(base, _EPT16)], dst_v)
        _zero_ref(zbuf, 128, fc)
        _zero_acc_rows(acc, zbuf, sid * _NROW, 5)
        plsc.subcore_barrier()

        def batch(b, _):
            e0 = b * 128
            for g in range(8):
                s16 = src_v[pl.ds(e0 + g * 16, 16)]
                gidx_v[pl.ds(g * 16, 16)] = s16 * 2 + _bc16(cid)
                sidx_v[pl.ds(g * 16, 16)] = dst_v[pl.ds(e0 + g * 16, 16)]
            pltpu.sync_copy(table_hbm.at[gidx_v], rows_v)
            pltpu.sync_copy(rows_v, acc.at[sidx_v], add=True)
            return 0

        lax.fori_loop(0, nb, batch, 0)
        plsc.subcore_barrier()

        def drain(k2, _):
            r0 = sid * _NROW + k2 * 128
            pltpu.sync_copy(acc.at[pl.ds(r0, 128)],
                            out_hbm.at[cid, pl.ds(r0, 128)])
            return 0

        lax.fori_loop(0, 5, drain, 0)

    return k


@functools.lru_cache(maxsize=None)
def _gat_ex_kernel(H):
    nb = _EPT32 // 128
    M = H * NPAD

    @functools.partial(
        pl.kernel, mesh=_SC_MESH,
        compiler_params=_SC_PARAMS,
        out_type=(jax.ShapeDtypeStruct((H, EP), jnp.float32),
                  jax.ShapeDtypeStruct((32, M), jnp.float32)),
        scratch_types=[
            pltpu.VMEM((_EPT32,), jnp.int32),
            pltpu.VMEM((_EPT32,), jnp.int32),
            pltpu.VMEM((NPAD,), jnp.float32),
            pltpu.VMEM((NPAD,), jnp.float32),
            pltpu.VMEM((16,), jnp.float32),
            pltpu.VMEM((128,), jnp.float32),
            pltpu.VMEM((M,), jnp.float32),
        ])
    def k(ast_hbm, adt_hbm, src_hbm, dst_hbm, shifts_hbm,
          ex_hbm, dn_hbm, src_v, dst_v, as_v, ad_v, sh_v, ex_v, priv):
        cid = lax.axis_index("c")
        sid = lax.axis_index("s")
        wid = sid * 2 + cid
        base = wid * _EPT32
        iota = lax.iota(jnp.int32, 16)
        pltpu.sync_copy(src_hbm.at[pl.ds(base, _EPT32)], src_v)
        pltpu.sync_copy(dst_hbm.at[pl.ds(base, _EPT32)], dst_v)
        pltpu.sync_copy(shifts_hbm, sh_v)
        zv = jnp.zeros((16,), jnp.float32)

        def zp(i, _):
            priv[pl.ds(i * 16, 16)] = zv
            return 0

        lax.fori_loop(0, M // 16, zp, 0)
        for h in range(H):
            pltpu.sync_copy(ast_hbm.at[h], as_v)
            pltpu.sync_copy(adt_hbm.at[h], ad_v)
            sh = _bc16(sh_v[pl.ds(0, 16)][h])

            def batch(b, _):
                e0 = b * 128
                for g in range(8):
                    s16 = src_v[pl.ds(e0 + g * 16, 16)]
                    d16 = dst_v[pl.ds(e0 + g * 16, 16)]
                    av = plsc.load_gather(as_v, [s16])
                    dv = plsc.load_gather(ad_v, [d16])
                    sv = av + dv
                    ev = jnp.maximum(sv, 0.2 * sv)
                    exv = jnp.exp(ev - sh)
                    gid = _bc16(base + e0 + g * 16) + iota
                    exv = jnp.where(gid < EV, exv, jnp.float32(0.0))
                    ex_v[pl.ds(g * 16, 16)] = exv
                    plsc.addupdate_scatter(priv, [d16 + h * NPAD], exv)
                pltpu.sync_copy(ex_v, ex_hbm.at[h, pl.ds(base + e0, 128)])
                return 0

            lax.fori_loop(0, nb, batch, 0)
        pltpu.sync_copy(priv, dn_hbm.at[wid])

    return k


@functools.lru_cache(maxsize=None)
def _gat_scatter_kernel(fc, nchunks, H):
    nb = _EPT16 // 128

    @functools.partial(
        pl.kernel, mesh=_SC_MESH, compiler_params=_SC_PARAMS,
        out_type=jax.ShapeDtypeStruct((nchunks, NPAD, fc), jnp.float32),
        scratch_types=[
            pltpu.VMEM((128,), jnp.int32),
            pltpu.VMEM((128,), jnp.int32),
            pltpu.VMEM((128,), jnp.int32),
            pltpu.VMEM((128,), jnp.float32),
            pltpu.VMEM((128, fc), jnp.float32),
            pltpu.VMEM_SHARED((NPAD, fc), jnp.float32),
        ])
    def k(table_hbm, src_hbm, dst_hbm, ex_hbm, out_hbm,
          sbuf, sidx_v, gidx_v, exb, rows_v, acc):
        cid = lax.axis_index("c")
        sid = lax.axis_index("s")
        base = sid * _EPT16
        for ch2 in range(nchunks // 2):
            ch = ch2 * 2 + cid
            head = ch * H // nchunks
            _zero_ref(rows_v, 128, fc)
            _zero_acc_rows(acc, rows_v, sid * _NROW, 5)
            plsc.subcore_barrier()

            def batch(b, _):
                e0 = base + b * 128
                pltpu.sync_copy(src_hbm.at[pl.ds(e0, 128)], sbuf)
                pltpu.sync_copy(dst_hbm.at[pl.ds(e0, 128)], sidx_v)
                pltpu.sync_copy(ex_hbm.at[head, pl.ds(e0, 128)], exb)
                for g in range(8):
                    s16 = sbuf[pl.ds(g * 16, 16)]
                    gidx_v[pl.ds(g * 16, 16)] = s16 * nchunks + _bc16(ch)
                pltpu.sync_copy(table_hbm.at[gidx_v], rows_v)

                def scale(g2, _):
                    w16 = exb[pl.ds(g2 * 16, 16)]
                    for l in range(16):
                        w = _bc16(w16[l])
                        r = g2 * 16 + l
                        for j in range(fc // 16):
                            rows_v[r, pl.ds(j * 16, 16)] = (
                                rows_v[r, pl.ds(j * 16, 16)] * w)
                    return 0

                lax.fori_loop(0, 8, scale, 0)
                pltpu.sync_copy(rows_v, acc.at[sidx_v], add=True)
                return 0

            lax.fori_loop(0, nb, batch, 0)
            plsc.subcore_barrier()

            def drain(k2, _):
                r0 = sid * _NROW + k2 * 128
                pltpu.sync_copy(acc.at[pl.ds(r0, 128)],
                                out_hbm.at[ch, pl.ds(r0, 128)])
                return 0

            lax.fori_loop(0, 5, drain, 0)
            plsc.subcore_barrier()

    return k


def _seg_deg(src_p, dst_p):
    out = _deg_kernel()(dst_p)
    return out[:, :N, 0]


def _seg_rows_gcn(hp, src_p, dst_p, f):
    fc = f // 2
    table = jnp.concatenate(
        [hp.reshape(N * 2, fc), jnp.zeros((2, fc), jnp.float32)], axis=0)
    outb = _gcn_scatter_kernel(fc)(table, src_p, dst_p)
    return outb[:, :N].transpose(1, 0, 2).reshape(N, f)


def _block_diag_attn(a, heads, f):
    return (jnp.eye(heads, dtype=jnp.float32)[:, None, :]
            * a[:, :, None]).reshape(heads * f, heads)


def kernel(x, params, edge_index):
    ar = jnp.arange(N, dtype=edge_index.dtype)
    src = jnp.concatenate([edge_index[0], ar])
    dst = jnp.concatenate([edge_index[1], ar])
    src_p = jnp.concatenate(
        [src, jnp.full((EP - EV,), N, dtype=src.dtype)])
    dst_p = jnp.concatenate(
        [dst, jnp.zeros((EP - EV,), dtype=dst.dtype)])

    deg_p = _seg_deg(src_p, dst_p)
    deg_corr = jnp.zeros((1, N), jnp.float32).at[0, 0].set(float(EP - EV))
    dis_row = _reduce_sc(deg_p, "rsqrt", deg_corr)
    dis = dis_row.reshape(N, 1)

    pg = params["gcn"]
    h = x
    for li, f in enumerate(GCN_DIMS):
        c = pg["convs"][li]
        sk = pg["skips"][li]
        bnp = pg["bns"][li]
        res = h if sk is None else _mm(h, sk["W"], sk["b"])
        hp = _mm(h, c["W"], jnp.zeros((f,), jnp.float32), row_scale=dis)
        scat = _seg_rows_gcn(hp, src_p, dst_p, f)
        h = _gcn_post(scat, dis, c["b"], bnp["g"], bnp["b"], res)
    h = _mm(h, pg["cls"][0]["W"], pg["cls"][0]["b"], relu=True)
    hg = _mm(h, pg["cls"][1]["W"], pg["cls"][1]["b"])

    pa = params["gat"]
    g = x
    for li, (f, hh) in enumerate(zip(GAT_DIMS, GAT_HEADS)):
        c = pa["convs"][li]
        bnp = pa["bns"][li]
        h2 = _mm(g, c["W"], jnp.zeros((hh * f,), jnp.float32))
        a_s = _mm(h2, _block_diag_attn(c["a_src"], hh, f),
                  jnp.zeros((hh,), jnp.float32))
        a_d = _mm(h2, _block_diag_attn(c["a_dst"], hh, f),
                  jnp.zeros((hh,), jnp.float32))
        shifts = _shift(a_s, a_d)

        a_st = jnp.zeros((hh, NPAD), jnp.float32).at[:, :N].set(a_s.T)
        a_dt = jnp.zeros((hh, NPAD), jnp.float32).at[:, :N].set(a_d.T)
        sh16 = jnp.zeros((16,), jnp.float32).at[:hh].set(shifts[0])
        exT, dn_p = _gat_ex_kernel(hh)(a_st, a_dt, src_p, dst_p, sh16)
        dn = dn_p.reshape(32, hh, NPAD)[:, :, :N].reshape(32, hh * N)
        rdn = _reduce_sc(dn, "recip").reshape(hh, N).T
        nchunks = max(2, hh)
        fc = hh * f // nchunks
        table = jnp.concatenate(
            [h2.reshape(N * nchunks, fc),
             jnp.zeros((nchunks, fc), jnp.float32)], axis=0)
        outb = _gat_scatter_kernel(fc, nchunks, hh)(table, src_p, dst_p, exT)
        scat = outb[:, :N].transpose(1, 0, 2).reshape(N, hh * f)
        g = _gat_post(scat, rdn, c["b"], bnp["g"], bnp["b"], hh)
    g = _mm(g, pa["cls"][0]["W"], pa["cls"][0]["b"], relu=True)
    ha = _mm(g, pa["cls"][1]["W"], pa["cls"][1]["b"])

    return _final(hg, ha, params["ens_w"].reshape(1, 2))


# speedup vs baseline: 17.3674x; 1.2582x over previous
import functools

import numpy as np
import jax
import jax.numpy as jnp
from jax import lax
from jax.experimental import pallas as pl
from jax.experimental.pallas import tpu as pltpu
from jax.experimental.pallas import tpu_sc as plsc

N = 10000
OUT_C = 2
BN_EPS = 1e-5
GCN_DIMS = [128, 64, 32]
GAT_DIMS = [128, 64, 32]
GAT_HEADS = [8, 4, 1]
BLK = 1000
_BN_INV = float(1.0 / np.sqrt(1.0 + BN_EPS))

E_RAW = 160000
EV = E_RAW + N
EP = 172032


def _mm(x, w, b, relu=False, row_scale=None):
    n, k = x.shape
    f = w.shape[1]
    scaled = row_scale is not None

    def body(*refs):
        if scaled:
            x_ref, w_ref, b_ref, s_ref, o_ref = refs
        else:
            x_ref, w_ref, b_ref, o_ref = refs
        y = jnp.dot(x_ref[...], w_ref[...],
                    preferred_element_type=jnp.float32) + b_ref[...]
        if scaled:
            y = y * s_ref[...]
        o_ref[...] = jnp.maximum(y, 0.0) if relu else y

    in_specs = [
        pl.BlockSpec((BLK, k), lambda i: (i, 0)),
        pl.BlockSpec((k, f), lambda i: (0, 0)),
        pl.BlockSpec((1, f), lambda i: (0, 0)),
    ]
    args = [x, w, b.reshape(1, f)]
    if scaled:
        in_specs.append(pl.BlockSpec((BLK, 1), lambda i: (i, 0)))
        args.append(row_scale)
    return pl.pallas_call(
        body,
        grid=(n // BLK,),
        in_specs=in_specs,
        out_specs=pl.BlockSpec((BLK, f), lambda i: (i, 0)),
        out_shape=jax.ShapeDtypeStruct((n, f), jnp.float32),
    )(*args)


def _gcn_post(scat, dis, bconv, g, bbn, res):
    f = scat.shape[1]

    def body(s_ref, d_ref, bc_ref, g_ref, bb_ref, r_ref, o_ref):
        t = (s_ref[...] * d_ref[...] + bc_ref[...]) * (g_ref[...] * _BN_INV) \
            + bb_ref[...]
        o_ref[...] = jnp.maximum(t, 0.0) + r_ref[...]

    return pl.pallas_call(
        body,
        grid=(N // BLK,),
        in_specs=[
            pl.BlockSpec((BLK, f), lambda i: (i, 0)),
            pl.BlockSpec((BLK, 1), lambda i: (i, 0)),
            pl.BlockSpec((1, f), lambda i: (0, 0)),
            pl.BlockSpec((1, f), lambda i: (0, 0)),
            pl.BlockSpec((1, f), lambda i: (0, 0)),
            pl.BlockSpec((BLK, f), lambda i: (i, 0)),
        ],
        out_specs=pl.BlockSpec((BLK, f), lambda i: (i, 0)),
        out_shape=jax.ShapeDtypeStruct((N, f), jnp.float32),
    )(scat, dis, bconv.reshape(1, f), g.reshape(1, f), bbn.reshape(1, f), res)


def _gat_post(scat, rdn, bias, g, bbn, heads):
    hf = scat.shape[1]
    f = hf // heads

    def body(s_ref, rd_ref, b_ref, g_ref, bb_ref, o_ref):
        s = s_ref[...].reshape(BLK, heads, f) * rd_ref[...].reshape(BLK, heads, 1)
        t = (s.reshape(BLK, hf) + b_ref[...]) * (g_ref[...] * _BN_INV) + bb_ref[...]
        o_ref[...] = jnp.maximum(t, 0.0)

    return pl.pallas_call(
        body,
        grid=(N // BLK,),
        in_specs=[
            pl.BlockSpec((BLK, hf), lambda i: (i, 0)),
            pl.BlockSpec((BLK, heads), lambda i: (i, 0)),
            pl.BlockSpec((1, hf), lambda i: (0, 0)),
            pl.BlockSpec((1, hf), lambda i: (0, 0)),
            pl.BlockSpec((1, hf), lambda i: (0, 0)),
        ],
        out_specs=pl.BlockSpec((BLK, hf), lambda i: (i, 0)),
        out_shape=jax.ShapeDtypeStruct((N, hf), jnp.float32),
    )(scat, rdn, bias.reshape(1, hf), g.reshape(1, hf), bbn.reshape(1, hf))


def _shift(a_s, a_d):
    h = a_s.shape[1]

    def body(s_ref, d_ref, o_ref):
        m = (jnp.max(s_ref[...], axis=0, keepdims=True)
             + jnp.max(d_ref[...], axis=0, keepdims=True))
        o_ref[...] = jnp.maximum(m, 0.2 * m)

    return pl.pallas_call(
        body, out_shape=jax.ShapeDtypeStruct((1, h), jnp.float32))(a_s, a_d)


def _reduce_sc(p, mode, corr=None):
    m = p.shape[1]
    has_corr = corr is not None

    def body(*refs):
        if has_corr:
            p_ref, c_ref, o_ref = refs
        else:
            p_ref, o_ref = refs
        s = jnp.sum(p_ref[...], axis=0, keepdims=True)
        if has_corr:
            s = s - c_ref[...]
        if mode == "rsqrt":
            r = lax.rsqrt(s)
        else:
            r = 1.0 / (s + 1e-16)
        o_ref[...] = r

    args = (p, corr) if has_corr else (p,)
    return pl.pallas_call(
        body, out_shape=jax.ShapeDtypeStruct((1, m), jnp.float32))(*args)


def _final(hg, ha, ens_w):
    def body(g_ref, a_ref, w_ref, o_ref):
        def lsm(t):
            mx = jnp.max(t, axis=1, keepdims=True)
            return t - (mx + jnp.log(jnp.sum(jnp.exp(t - mx), axis=1,
                                             keepdims=True)))
        wv = w_ref[...]
        wv = jnp.exp(wv - jnp.max(wv))
        wv = wv / jnp.sum(wv)
        o_ref[...] = wv[:, 0:1] * lsm(g_ref[...]) + wv[:, 1:2] * lsm(a_ref[...])

    return pl.pallas_call(
        body, out_shape=jax.ShapeDtypeStruct((N, OUT_C), jnp.float32),
    )(hg, ha, ens_w)


_SC_MESH = plsc.VectorSubcoreMesh(core_axis_name="c", subcore_axis_name="s")
_SC_PARAMS = pltpu.CompilerParams(needs_layout_passes=False,
                                  use_tc_tiling_on_sc=False)
_EPT16 = EP // 16
_EPT32 = EP // 32
NPAD = 10240
_NROW = NPAD // 16


def _bc16(x):
    return lax.broadcast_in_dim(x, (16,), ())


def _zero_ref(ref, nrows, w):
    z = jnp.zeros((16,), jnp.float32)

    def bd(r, _):
        for j in range(w // 16):
            ref[r, pl.ds(j * 16, 16)] = z
        return 0

    lax.fori_loop(0, nrows, bd, 0)


def _zero_acc_rows(acc, zbuf, row0, ncopies):
    def bd(k, _):
        pltpu.sync_copy(zbuf, acc.at[pl.ds(row0 + k * 128, 128)])
        return 0

    lax.fori_loop(0, ncopies, bd, 0)


@functools.lru_cache(maxsize=None)
def _deg_kernel():
    @functools.partial(
        pl.kernel, mesh=_SC_MESH, compiler_params=_SC_PARAMS,
        out_type=jax.ShapeDtypeStruct((2, NPAD, 16), jnp.float32),
        scratch_types=[
            pltpu.VMEM((_EPT32,), jnp.int32),
            pltpu.VMEM((128, 16), jnp.float32),
            pltpu.VMEM((128, 16), jnp.float32),
            pltpu.VMEM((128,), jnp.int32),
            pltpu.VMEM_SHARED((NPAD, 16), jnp.float32),
        ])
    def k(dst_hbm, out_hbm, dst_v, buf, zbuf, sidx_v, acc):
        cid = lax.axis_index("c")
        sid = lax.axis_index("s")
        wid = sid * 2 + cid
        iota = lax.iota(jnp.int32, 16)
        pltpu.sync_copy(dst_hbm.at[pl.ds(wid * _EPT32, _EPT32)], dst_v)
        one0 = jnp.where(iota < 1, jnp.float32(1.0), jnp.float32(0.0))
        def fill(r, _):
            buf[r, :] = one0
            return 0
        lax.fori_loop(0, 128, fill, 0)
        _zero_ref(zbuf, 128, 16)
        _zero_acc_rows(acc, zbuf, sid * _NROW, 5)
        plsc.subcore_barrier()

        def batch(b, _):
            e0 = b * 128
            for g in range(8):
                d16 = dst_v[pl.ds(e0 + g * 16, 16)]
                sidx_v[pl.ds(g * 16, 16)] = d16
            pltpu.sync_copy(buf, acc.at[sidx_v], add=True)
            return 0

        lax.fori_loop(0, _EPT32 // 128, batch, 0)
        plsc.subcore_barrier()

        def drain(k2, _):
            r0 = sid * _NROW + k2 * 128
            pltpu.sync_copy(acc.at[pl.ds(r0, 128)],
                            out_hbm.at[cid, pl.ds(r0, 128)])
            return 0

        lax.fori_loop(0, 5, drain, 0)

    return k


@functools.lru_cache(maxsize=None)
def _gcn_scatter_kernel(fc):
    nb = _EPT16 // 128

    @functools.partial(
        pl.kernel, mesh=_SC_MESH, compiler_params=_SC_PARAMS,
        out_type=jax.ShapeDtypeStruct((2, NPAD, fc), jnp.float32),
        scratch_types=[
            pltpu.VMEM((_EPT16,), jnp.int32),
            pltpu.VMEM((_EPT16,), jnp.int32),
            pltpu.VMEM((128,), jnp.int32),
            pltpu.VMEM((128,), jnp.int32),
            pltpu.VMEM((128,), jnp.int32),
            pltpu.VMEM((128,), jnp.int32),
            pltpu.VMEM((128, fc), jnp.float32),
            pltpu.VMEM((128, fc), jnp.float32),
            pltpu.VMEM_SHARED((NPAD, fc), jnp.float32),
            pltpu.SemaphoreType.DMA,
            pltpu.SemaphoreType.DMA,
            pltpu.SemaphoreType.DMA,
            pltpu.SemaphoreType.DMA,
        ])
    def k(table_hbm, src_hbm, dst_hbm, out_hbm,
          src_v, dst_v, gidx0, gidx1, sidx0, sidx1, rows0, rows1, acc,
          sg0, sg1, ss0, ss1):
        cid = lax.axis_index("c")
        sid = lax.axis_index("s")
        base = sid * _EPT16
        gidx = (gidx0, gidx1)
        sidx = (sidx0, sidx1)
        rows = (rows0, rows1)
        sg = (sg0, sg1)
        ss = (ss0, ss1)
        pltpu.sync_copy(src_hbm.at[pl.ds(base, _EPT16)], src_v)
        pltpu.sync_copy(dst_hbm.at[pl.ds(base, _EPT16)], dst_v)
        _zero_ref(rows0, 128, fc)
        _zero_acc_rows(acc, rows0, sid * _NROW, 5)
        plsc.subcore_barrier()

        def prep(b, par):
            e0 = b * 128
            for g in range(8):
                s16 = src_v[pl.ds(e0 + g * 16, 16)]
                gidx[par][pl.ds(g * 16, 16)] = s16 * 2 + _bc16(cid)
                sidx[par][pl.ds(g * 16, 16)] = dst_v[pl.ds(e0 + g * 16, 16)]

        def half(b, cur, nxt):
            pltpu.make_async_copy(
                table_hbm.at[gidx[cur]], rows[cur], sg[cur]).wait()

            @pl.when(b + 1 < nb)
            def _():
                prep(b + 1, nxt)

                @pl.when(b >= 1)
                def _():
                    pltpu.make_async_copy(
                        rows[nxt], acc.at[sidx[nxt]], ss[nxt]).wait()

                pltpu.async_copy(
                    table_hbm.at[gidx[nxt]], rows[nxt], sg[nxt])

            pltpu.async_copy(rows[cur], acc.at[sidx[cur]], ss[cur],
                             add=True)

        prep(0, 0)
        pltpu.async_copy(table_hbm.at[gidx[0]], rows[0], sg[0])

        def pair(i, _):
            half(2 * i, 0, 1)
            half(2 * i + 1, 1, 0)
            return 0

        lax.fori_loop(0, nb // 2, pair, 0)
        pltpu.make_async_copy(rows[0], acc.at[sidx[0]], ss[0]).wait()
        pltpu.make_async_copy(rows[1], acc.at[sidx[1]], ss[1]).wait()
        plsc.subcore_barrier()

        def drain(k2, _):
            r0 = sid * _NROW + k2 * 128
            pltpu.sync_copy(acc.at[pl.ds(r0, 128)],
                            out_hbm.at[cid, pl.ds(r0, 128)])
            return 0

        lax.fori_loop(0, 5, drain, 0)

    return k


@functools.lru_cache(maxsize=None)
def _gat_ex_kernel(H):
    nb = _EPT32 // 128
    M = H * NPAD

    @functools.partial(
        pl.kernel, mesh=_SC_MESH,
        compiler_params=_SC_PARAMS,
        out_type=(jax.ShapeDtypeStruct((H, EP), jnp.float32),
                  jax.ShapeDtypeStruct((32, M), jnp.float32)),
        scratch_types=[
            pltpu.VMEM((_EPT32,), jnp.int32),
            pltpu.VMEM((_EPT32,), jnp.int32),
            pltpu.VMEM((NPAD,), jnp.float32),
            pltpu.VMEM((NPAD,), jnp.float32),
            pltpu.VMEM((16,), jnp.float32),
            pltpu.VMEM((128,), jnp.float32),
            pltpu.VMEM((M,), jnp.float32),
        ])
    def k(ast_hbm, adt_hbm, src_hbm, dst_hbm, shifts_hbm,
          ex_hbm, dn_hbm, src_v, dst_v, as_v, ad_v, sh_v, ex_v, priv):
        cid = lax.axis_index("c")
        sid = lax.axis_index("s")
        wid = sid * 2 + cid
        base = wid * _EPT32
        iota = lax.iota(jnp.int32, 16)
        pltpu.sync_copy(src_hbm.at[pl.ds(base, _EPT32)], src_v)
        pltpu.sync_copy(dst_hbm.at[pl.ds(base, _EPT32)], dst_v)
        pltpu.sync_copy(shifts_hbm, sh_v)
        zv = jnp.zeros((16,), jnp.float32)

        def zp(i, _):
            priv[pl.ds(i * 16, 16)] = zv
            return 0

        lax.fori_loop(0, M // 16, zp, 0)
        for h in range(H):
            pltpu.sync_copy(ast_hbm.at[h], as_v)
            pltpu.sync_copy(adt_hbm.at[h], ad_v)
            sh = _bc16(sh_v[pl.ds(0, 16)][h])

            def batch(b, _):
                e0 = b * 128
                for g in range(8):
                    s16 = src_v[pl.ds(e0 + g * 16, 16)]
                    d16 = dst_v[pl.ds(e0 + g * 16, 16)]
                    av = plsc.load_gather(as_v, [s16])
                    dv = plsc.load_gather(ad_v, [d16])
                    sv = av + dv
                    ev = jnp.maximum(sv, 0.2 * sv)
                    exv = jnp.exp(ev - sh)
                    gid = _bc16(base + e0 + g * 16) + iota
                    exv = jnp.where(gid < EV, exv, jnp.float32(0.0))
                    ex_v[pl.ds(g * 16, 16)] = exv
                    plsc.addupdate_scatter(priv, [d16 + h * NPAD], exv)
                pltpu.sync_copy(ex_v, ex_hbm.at[h, pl.ds(base + e0, 128)])
                return 0

            lax.fori_loop(0, nb, batch, 0)
        pltpu.sync_copy(priv, dn_hbm.at[wid])

    return k


@functools.lru_cache(maxsize=None)
def _gat_scatter_kernel(fc, nchunks, H):
    nb = _EPT16 // 128

    @functools.partial(
        pl.kernel, mesh=_SC_MESH, compiler_params=_SC_PARAMS,
        out_type=jax.ShapeDtypeStruct((nchunks, NPAD, fc), jnp.float32),
        scratch_types=[
            pltpu.VMEM((128,), jnp.int32),
            pltpu.VMEM((128,), jnp.int32),
            pltpu.VMEM((128,), jnp.int32),
            pltpu.VMEM((128,), jnp.int32),
            pltpu.VMEM((128,), jnp.int32),
            pltpu.VMEM((128,), jnp.float32),
            pltpu.VMEM((128, fc), jnp.float32),
            pltpu.VMEM((128, fc), jnp.float32),
            pltpu.VMEM_SHARED((NPAD, fc), jnp.float32),
            pltpu.SemaphoreType.DMA,
            pltpu.SemaphoreType.DMA,
            pltpu.SemaphoreType.DMA,
            pltpu.SemaphoreType.DMA,
        ])
    def k(table_hbm, src_hbm, dst_hbm, ex_hbm, out_hbm,
          sbuf, gidx0, gidx1, sidx0, sidx1, exb, rows0, rows1, acc,
          sg0, sg1, ss0, ss1):
        cid = lax.axis_index("c")
        sid = lax.axis_index("s")
        base = sid * _EPT16
        gidx = (gidx0, gidx1)
        sidx = (sidx0, sidx1)
        rows = (rows0, rows1)
        sg = (sg0, sg1)
        ss = (ss0, ss1)
        for ch2 in range(nchunks // 2):
            ch = ch2 * 2 + cid
            head = ch * H // nchunks
            _zero_ref(rows0, 128, fc)
            _zero_acc_rows(acc, rows0, sid * _NROW, 5)
            plsc.subcore_barrier()

            def prep(b, par):
                e0 = base + b * 128
                pltpu.sync_copy(src_hbm.at[pl.ds(e0, 128)], sbuf)
                pltpu.sync_copy(dst_hbm.at[pl.ds(e0, 128)], sidx[par])
                for g in range(8):
                    s16 = sbuf[pl.ds(g * 16, 16)]
                    gidx[par][pl.ds(g * 16, 16)] = (
                        s16 * nchunks + _bc16(ch))

            def half(b, cur, nxt):
                pltpu.make_async_copy(
                    table_hbm.at[gidx[cur]], rows[cur], sg[cur]).wait()

                @pl.when(b + 1 < nb)
                def _():
                    prep(b + 1, nxt)

                    @pl.when(b >= 1)
                    def _():
                        pltpu.make_async_copy(
                            rows[nxt], acc.at[sidx[nxt]], ss[nxt]).wait()

                    pltpu.async_copy(
                        table_hbm.at[gidx[nxt]], rows[nxt], sg[nxt])

                e0 = base + b * 128
                pltpu.sync_copy(ex_hbm.at[head, pl.ds(e0, 128)], exb)

                def scale(g2, _):
                    w16 = exb[pl.ds(g2 * 16, 16)]
                    for l in range(16):
                        w = _bc16(w16[l])
                        r = g2 * 16 + l
                        for j in range(fc // 16):
                            rows[cur][r, pl.ds(j * 16, 16)] = (
                                rows[cur][r, pl.ds(j * 16, 16)] * w)
                    return 0

                lax.fori_loop(0, 8, scale, 0)
                pltpu.async_copy(rows[cur], acc.at[sidx[cur]], ss[cur],
                                 add=True)

            prep(0, 0)
            pltpu.async_copy(table_hbm.at[gidx[0]], rows[0], sg[0])

            def pair(i, _):
                half(2 * i, 0, 1)
                half(2 * i + 1, 1, 0)
                return 0

            lax.fori_loop(0, nb // 2, pair, 0)
            pltpu.make_async_copy(rows[0], acc.at[sidx[0]], ss[0]).wait()
            pltpu.make_async_copy(rows[1], acc.at[sidx[1]], ss[1]).wait()
            plsc.subcore_barrier()

            def drain(k2, _):
                r0 = sid * _NROW + k2 * 128
                pltpu.sync_copy(acc.at[pl.ds(r0, 128)],
                                out_hbm.at[ch, pl.ds(r0, 128)])
                return 0

            lax.fori_loop(0, 5, drain, 0)
            plsc.subcore_barrier()

    return k


def _seg_deg(src_p, dst_p):
    out = _deg_kernel()(dst_p)
    return out[:, :N, 0]


def _seg_rows_gcn(hp, src_p, dst_p, f):
    fc = f // 2
    table = jnp.concatenate(
        [hp.reshape(N * 2, fc), jnp.zeros((2, fc), jnp.float32)], axis=0)
    outb = _gcn_scatter_kernel(fc)(table, src_p, dst_p)
    return outb[:, :N].transpose(1, 0, 2).reshape(N, f)


def _block_diag_attn(a, heads, f):
    return (jnp.eye(heads, dtype=jnp.float32)[:, None, :]
            * a[:, :, None]).reshape(heads * f, heads)


def kernel(x, params, edge_index):
    ar = jnp.arange(N, dtype=edge_index.dtype)
    src = jnp.concatenate([edge_index[0], ar])
    dst = jnp.concatenate([edge_index[1], ar])
    src_p = jnp.concatenate(
        [src, jnp.full((EP - EV,), N, dtype=src.dtype)])
    dst_p = jnp.concatenate(
        [dst, jnp.zeros((EP - EV,), dtype=dst.dtype)])

    deg_p = _seg_deg(src_p, dst_p)
    deg_corr = jnp.zeros((1, N), jnp.float32).at[0, 0].set(float(EP - EV))
    dis_row = _reduce_sc(deg_p, "rsqrt", deg_corr)
    dis = dis_row.reshape(N, 1)

    pg = params["gcn"]
    h = x
    for li, f in enumerate(GCN_DIMS):
        c = pg["convs"][li]
        sk = pg["skips"][li]
        bnp = pg["bns"][li]
        res = h if sk is None else _mm(h, sk["W"], sk["b"])
        hp = _mm(h, c["W"], jnp.zeros((f,), jnp.float32), row_scale=dis)
        scat = _seg_rows_gcn(hp, src_p, dst_p, f)
        h = _gcn_post(scat, dis, c["b"], bnp["g"], bnp["b"], res)
    h = _mm(h, pg["cls"][0]["W"], pg["cls"][0]["b"], relu=True)
    hg = _mm(h, pg["cls"][1]["W"], pg["cls"][1]["b"])

    pa = params["gat"]
    g = x
    for li, (f, hh) in enumerate(zip(GAT_DIMS, GAT_HEADS)):
        c = pa["convs"][li]
        bnp = pa["bns"][li]
        h2 = _mm(g, c["W"], jnp.zeros((hh * f,), jnp.float32))
        a_s = _mm(h2, _block_diag_attn(c["a_src"], hh, f),
                  jnp.zeros((hh,), jnp.float32))
        a_d = _mm(h2, _block_diag_attn(c["a_dst"], hh, f),
                  jnp.zeros((hh,), jnp.float32))
        shifts = _shift(a_s, a_d)

        a_st = jnp.zeros((hh, NPAD), jnp.float32).at[:, :N].set(a_s.T)
        a_dt = jnp.zeros((hh, NPAD), jnp.float32).at[:, :N].set(a_d.T)
        sh16 = jnp.zeros((16,), jnp.float32).at[:hh].set(shifts[0])
        exT, dn_p = _gat_ex_kernel(hh)(a_st, a_dt, src_p, dst_p, sh16)
        dn = dn_p.reshape(32, hh, NPAD)[:, :, :N].reshape(32, hh * N)
        rdn = _reduce_sc(dn, "recip").reshape(hh, N).T
        nchunks = max(2, hh)
        fc = hh * f // nchunks
        table = jnp.concatenate(
            [h2.reshape(N * nchunks, fc),
             jnp.zeros((nchunks, fc), jnp.float32)], axis=0)
        outb = _gat_scatter_kernel(fc, nchunks, hh)(table, src_p, dst_p, exT)
        scat = outb[:, :N].transpose(1, 0, 2).reshape(N, hh * f)
        g = _gat_post(scat, rdn, c["b"], bnp["g"], bnp["b"], hh)
    g = _mm(g, pa["cls"][0]["W"], pa["cls"][0]["b"], relu=True)
    ha = _mm(g, pa["cls"][1]["W"], pa["cls"][1]["b"])

    return _final(hg, ha, params["ens_w"].reshape(1, 2))
